# Initial kernel scaffold; baseline (speedup 1.0000x reference)
import jax
import jax.numpy as jnp
from jax.experimental import pallas as pl


def kernel(x, conv0_w, conv0_scale, conv0_bias, conv1_w, conv1_scale, conv1_bias, conv2_w, conv2_scale, conv2_bias, conv3_w, conv3_scale, conv3_bias, conv4_w, conv4_scale, conv4_bias, fc0_w, fc0_b, fc1_w, fc1_b, fc2_w, fc2_b):
    raise NotImplementedError("write your pallas kernel here")


# speedup vs baseline: 4.8743x; 4.8743x over previous
import functools

import jax
import jax.numpy as jnp
from jax.experimental import pallas as pl
from jax.experimental.pallas import tpu as pltpu

_VMEM_LIMIT = 48 * 1024 * 1024

_STAGES = [
    dict(h=64, w=64, wp=66, cin=128, cout=128, g=1, r_in=4432, r_out=1200, wp_out=34, pad_out=True),
    dict(h=32, w=32, wp=34, cin=128, cout=256, g=1, r_in=1200, r_out=352, wp_out=18, pad_out=True),
    dict(h=16, w=16, wp=18, cin=256, cout=512, g=4, r_in=352, r_out=112, wp_out=10, pad_out=True),
    dict(h=8, w=8, wp=10, cin=512, cout=512, g=8, r_in=112, r_out=16, wp_out=4, pad_out=False),
]

_TAP_PAIRS = [(0, 1), (2, 3), (4, 5), (6, 7), (8, 8)]


def _pool_write(scrs, o_ref, *, g, h, wp, wp_out, pad_out):
    h2, w2 = h // 2, (wp - 2) // 2
    if pad_out:
        o_ref[...] = jnp.zeros_like(o_ref)
    for m in range(g):
        for i in range(h2):
            src = m * h * wp + 2 * i * wp
            if pad_out:
                dst = (1 + i) * wp_out + 1
            else:
                dst = i * w2
            for k, scr in enumerate(scrs):
                o_ref[m, pl.ds(dst, w2), k * 128:(k + 1) * 128] = (
                    scr[pl.ds(src, w2, 2), :].astype(o_ref.dtype))


def _conv_stage_kernel(x_ref, w_ref, s_ref, b_ref, o_ref, *scrs,
                       g, h, wp, cin, cout, wp_out, pad_out):
    r = h * wp
    acc = jnp.zeros((g * r, cout), jnp.float32)
    for gi, (ta, tb) in enumerate(_TAP_PAIRS):
        offa = (ta // 3) * wp + ta % 3
        offb = (tb // 3) * wp + tb % 3
        if g == 1:
            a = x_ref[0, offa:offa + r, :]
            b = x_ref[0, offb:offb + r, :]
        else:
            a = jnp.concatenate([x_ref[m, offa:offa + r, :] for m in range(g)], axis=0)
            b = jnp.concatenate([x_ref[m, offb:offb + r, :] for m in range(g)], axis=0)
        lhs = jnp.concatenate([a, b], axis=1)
        acc = acc + jnp.dot(lhs, w_ref[gi], preferred_element_type=jnp.float32)
    y = jnp.maximum(acc * s_ref[...] + b_ref[...], 0.0)
    rows = g * r
    y = jnp.maximum(y, pltpu.roll(y, shift=rows - 1, axis=0))
    y = jnp.maximum(y, pltpu.roll(y, shift=rows - wp, axis=0))
    for k, scr in enumerate(scrs):
        scr[...] = y[:, k * 128:(k + 1) * 128]
    _pool_write(scrs, o_ref, g=g, h=h, wp=wp, wp_out=wp_out, pad_out=pad_out)


def _conv_stage(x, wg, scale, bias, *, h, w, wp, cin, cout, g, r_in, r_out,
                wp_out, pad_out):
    n = x.shape[0]
    r = h * wp
    return pl.pallas_call(
        functools.partial(_conv_stage_kernel, g=g, h=h, wp=wp, cin=cin,
                          cout=cout, wp_out=wp_out, pad_out=pad_out),
        out_shape=jax.ShapeDtypeStruct((n, r_out, cout), jnp.bfloat16),
        grid=(n // g,),
        in_specs=[
            pl.BlockSpec((g, r_in, cin), lambda i: (i, 0, 0)),
            pl.BlockSpec(wg.shape, lambda i: (0, 0, 0)),
            pl.BlockSpec((1, cout), lambda i: (0, 0)),
            pl.BlockSpec((1, cout), lambda i: (0, 0)),
        ],
        out_specs=pl.BlockSpec((g, r_out, cout), lambda i: (i, 0, 0)),
        scratch_shapes=[pltpu.VMEM((g * r, 128), jnp.float32)
                        for _ in range(cout // 128)],
        compiler_params=pltpu.CompilerParams(
            dimension_semantics=("parallel",),
            vmem_limit_bytes=_VMEM_LIMIT),
    )(x, wg, scale, bias)


def _conv1_kernel(p_ref, w_ref, s_ref, b_ref, o_ref, y_scr, *, hw, wp_out):
    o_ref[...] = jnp.zeros_like(o_ref)
    for blk in range(4):
        rows = 32 * hw
        acc = jnp.dot(p_ref[0, blk * rows:(blk + 1) * rows, :], w_ref[...],
                      preferred_element_type=jnp.float32)
        y = jnp.maximum(acc * s_ref[...] + b_ref[...], 0.0)
        y = jnp.maximum(y, pltpu.roll(y, shift=rows - 1, axis=0))
        y = jnp.maximum(y, pltpu.roll(y, shift=rows - hw, axis=0))
        y_scr[...] = y
        for i in range(16):
            dst = (1 + 16 * blk + i) * wp_out + 1
            o_ref[0, pl.ds(dst, hw // 2), :] = y_scr[pl.ds(2 * i * hw, hw // 2, 2), :].astype(o_ref.dtype)


def _conv1(pats, w, scale, bias, *, r_out, wp_out):
    n = pats.shape[0]
    return pl.pallas_call(
        functools.partial(_conv1_kernel, hw=128, wp_out=wp_out),
        out_shape=jax.ShapeDtypeStruct((n, r_out, 128), jnp.bfloat16),
        grid=(n,),
        in_specs=[
            pl.BlockSpec((1, 128 * 128, 27), lambda i: (i, 0, 0)),
            pl.BlockSpec((27, 128), lambda i: (0, 0)),
            pl.BlockSpec((1, 128), lambda i: (0, 0)),
            pl.BlockSpec((1, 128), lambda i: (0, 0)),
        ],
        out_specs=pl.BlockSpec((1, r_out, 128), lambda i: (i, 0, 0)),
        scratch_shapes=[pltpu.VMEM((32 * 128, 128), jnp.float32)],
        compiler_params=pltpu.CompilerParams(
            dimension_semantics=("parallel",),
            vmem_limit_bytes=_VMEM_LIMIT),
    )(pats, w, scale, bias)


def _fc1_kernel(x_ref, w_ref, b_ref, o_ref):
    y = jnp.dot(x_ref[...], w_ref[...], preferred_element_type=jnp.float32)
    o_ref[...] = jnp.maximum(y + b_ref[...], 0.0).astype(jnp.bfloat16)


def _fc23_kernel(h_ref, w1_ref, b1_ref, w2_ref, b2_ref, o_ref):
    y = jnp.dot(h_ref[...], w1_ref[...], preferred_element_type=jnp.float32)
    h2 = jnp.maximum(y + b1_ref[...], 0.0).astype(jnp.bfloat16)
    o_ref[...] = jnp.dot(h2, w2_ref[...], preferred_element_type=jnp.float32) + b2_ref[...]


def _fc_head(feats, fc0_w, fc0_b, fc1_w, fc1_b, fc2_w, fc2_b):
    m, k = feats.shape
    h1 = pl.pallas_call(
        _fc1_kernel,
        out_shape=jax.ShapeDtypeStruct((m, 1024), jnp.bfloat16),
        grid=(2,),
        in_specs=[
            pl.BlockSpec((m, k), lambda j: (0, 0)),
            pl.BlockSpec((k, 512), lambda j: (0, j)),
            pl.BlockSpec((1, 512), lambda j: (0, j)),
        ],
        out_specs=pl.BlockSpec((m, 512), lambda j: (0, j)),
        compiler_params=pltpu.CompilerParams(
            dimension_semantics=("parallel",),
            vmem_limit_bytes=_VMEM_LIMIT),
    )(feats, fc0_w, fc0_b)
    return pl.pallas_call(
        _fc23_kernel,
        out_shape=jax.ShapeDtypeStruct((m, 128), jnp.float32),
        grid=(1,),
        in_specs=[
            pl.BlockSpec((m, 1024), lambda j: (0, 0)),
            pl.BlockSpec((1024, 512), lambda j: (0, 0)),
            pl.BlockSpec((1, 512), lambda j: (0, 0)),
            pl.BlockSpec((512, 128), lambda j: (0, 0)),
            pl.BlockSpec((1, 128), lambda j: (0, 0)),
        ],
        out_specs=pl.BlockSpec((m, 128), lambda j: (0, 0)),
        compiler_params=pltpu.CompilerParams(
            vmem_limit_bytes=_VMEM_LIMIT),
    )(h1, fc1_w, fc1_b, fc2_w, fc2_b)


def _pair_taps(w):
    cin, cout = w.shape[2], w.shape[3]
    wf = w.reshape(9, cin, cout)
    zero = jnp.zeros((cin, cout), w.dtype)
    groups = [jnp.concatenate([wf[a], wf[b] if b != a else zero], axis=0)
              for a, b in _TAP_PAIRS]
    return jnp.stack(groups, axis=0)


def _im2col_l1(x):
    n = x.shape[0]
    xh = jnp.transpose(x, (0, 2, 3, 1)).astype(jnp.bfloat16)
    xp = jnp.pad(xh, ((0, 0), (1, 1), (1, 1), (0, 0)))
    pats = jnp.stack([xp[:, dy:dy + 128, dx:dx + 128, :]
                      for dy in range(3) for dx in range(3)], axis=3)
    return pats.reshape(n, 128 * 128, 27)


def kernel(x, conv0_w, conv0_scale, conv0_bias, conv1_w, conv1_scale, conv1_bias,
           conv2_w, conv2_scale, conv2_bias, conv3_w, conv3_scale, conv3_bias,
           conv4_w, conv4_scale, conv4_bias, fc0_w, fc0_b, fc1_w, fc1_b,
           fc2_w, fc2_b):
    n = x.shape[0]
    pats = _im2col_l1(x)
    s0 = _STAGES[0]
    h = _conv1(pats, conv0_w, conv0_scale, conv0_bias,
               r_out=s0["r_in"], wp_out=s0["wp"])
    conv_ws = [conv1_w, conv2_w, conv3_w, conv4_w]
    conv_ss = [conv1_scale, conv2_scale, conv3_scale, conv4_scale]
    conv_bs = [conv1_bias, conv2_bias, conv3_bias, conv4_bias]
    for st, w, s, b in zip(_STAGES, conv_ws, conv_ss, conv_bs):
        h = _conv_stage(h, _pair_taps(w), s, b, **st)
    feats = h.reshape(n, 16 * 512)
    logits = _fc_head(feats, fc0_w, fc0_b, fc1_w, fc1_b, fc2_w, fc2_b)
    return logits[:, :11]
